# baseline (speedup 1.0000x reference)
import math

import jax
import jax.numpy as jnp
from jax import lax
from jax.experimental import pallas as pl
from jax.experimental.pallas import tpu as pltpu

F32 = jnp.float32
NZ, NTOK, SEQ = 50, 128, 258
NHEAD, DHEAD, NHID = 5, 10, 512
ENC_CHS = [1, 16, 32, 64, 64, 64, 50]
ARC_S = 30.0
COS_M, SIN_M = math.cos(0.5), math.sin(0.5)
TH = math.cos(math.pi - 0.5)
MM = math.sin(math.pi - 0.5) * 0.5
INV_SQRT_D = 1.0 / math.sqrt(float(DHEAD))

EB = 16
TB = 8
DB = 8


def _mm(a, b):
    return lax.dot_general(a, b, (((a.ndim - 1,), (0,)), ((), ())),
                           preferred_element_type=F32)


def _sd(p):
    z = jnp.zeros((p.shape[0], 1, p.shape[2]), F32)
    return jnp.concatenate([z, p[:, :-1, :]], axis=1)


def _su(p):
    z = jnp.zeros((p.shape[0], 1, p.shape[2]), F32)
    return jnp.concatenate([p[:, 1:, :], z], axis=1)


def _enc_body(x_ref, w0, w1, w2, w3, w4, w5, out_ref):
    ws = (w0, w1, w2, w3, w4, w5)
    a = x_ref[...]
    c = 1
    for li in range(6):
        w = ws[li][0]
        s2 = a.shape[2] // (2 * c)
        aext = jnp.concatenate(
            [_sd(a[:, :, -c:]), a, _su(a[:, :, :c])], axis=-1)
        pieces = [_mm(aext[:, :, 2 * q * c:(2 * q + 4) * c], w)
                  for q in range(s2)]
        a = pieces[0] if s2 == 1 else jnp.concatenate(pieces, axis=-1)
        if li < 5:
            a = jnp.where(a > 0, a, 0.2 * a)
        c = w.shape[1]
    out_ref[...] = a


def _run_encoders(x_sig, x_freq, params):
    xin = jnp.concatenate([x_sig.reshape(-1, 8192),
                           x_freq.reshape(-1, 8192)],
                          axis=0).reshape(512, 128, 64)
    wstk = []
    for i in range(6):
        co = ENC_CHS[i + 1]
        wstk.append(jnp.stack(
            [params['enc_sig'][i].transpose(2, 1, 0).reshape(-1, co),
             params['enc_freq'][i].transpose(2, 1, 0).reshape(-1, co)]))
    nb = 256 // EB
    w_specs = [pl.BlockSpec((1,) + w.shape[1:], lambda s, i: (s, 0, 0))
               for w in wstk]
    return pl.pallas_call(
        _enc_body,
        grid=(2, nb),
        in_specs=[pl.BlockSpec((EB, 128, 64), lambda s, i: (s * nb + i, 0, 0))]
                 + w_specs,
        out_specs=pl.BlockSpec((EB, 128, 50), lambda s, i: (s * nb + i, 0, 0)),
        out_shape=jax.ShapeDtypeStruct((512, 128, 50), F32),
        compiler_params=pltpu.CompilerParams(
            dimension_semantics=("arbitrary", "arbitrary"),
            vmem_limit_bytes=100 * 1024 * 1024),
        name="enc6",
    )(xin, *wstk)


def _ln(x, g, bb):
    m = jnp.mean(x, -1, keepdims=True)
    d = x - m
    v = jnp.mean(d * d, -1, keepdims=True)
    return d * lax.rsqrt(v + 1e-5) * g + bb


def _attn_one(qrows, krows, vrows):
    outs = []
    for h in range(NHEAD):
        q = qrows[:, h * DHEAD:(h + 1) * DHEAD]
        k = krows[:, h * DHEAD:(h + 1) * DHEAD]
        v = vrows[:, h * DHEAD:(h + 1) * DHEAD]
        s = lax.dot_general(q, k, (((1,), (1,)), ((), ())),
                            preferred_element_type=F32) * INV_SQRT_D
        e = jnp.exp(s - jnp.max(s, -1, keepdims=True))
        o = lax.dot_general(e, v, (((1,), (0,)), ((), ())),
                            preferred_element_type=F32)
        outs.append(o / jnp.sum(e, -1, keepdims=True))
    return jnp.concatenate(outs, axis=-1)


def _arc(v, wnt, label):
    n = jnp.sqrt(jnp.sum(v * v, -1, keepdims=True))
    xn = v / (n + 1e-12)
    cos = _mm(xn, wnt)
    sin = jnp.sqrt(jnp.clip(1.0 - cos * cos, 0.0, 1.0))
    phi = cos * COS_M - sin * SIN_M
    phi = jnp.where(cos > TH, phi, cos - MM)
    col = lax.broadcasted_iota(jnp.int32, cos.shape, 1)
    return ARC_S * jnp.where(col == label, phi, cos)


def _tf_body(sig_ref, frq_ref, base_ref, wqkv, bqkv, wo, bo,
             w1, b1, w2, b2, lnp, wnt_ref, as_ref, af_ref):
    b = sig_ref.shape[0]
    z1 = jnp.zeros((b, 1, NZ), F32)
    x = (jnp.concatenate([z1, sig_ref[...], z1, frq_ref[...]], axis=1)
         + base_ref[...][None])
    for l in range(2):
        qkv = _mm(x, wqkv[l]) + bqkv[l]
        ao = jnp.stack([_attn_one(qkv[i, :, :50], qkv[i, :, 50:100],
                                  qkv[i, :, 100:]) for i in range(b)])
        x = _ln(x + _mm(ao, wo[l]) + bo[l], lnp[l][0:1], lnp[l][1:2])
        ff = _mm(jnp.maximum(_mm(x, w1[l]) + b1[l], 0.0), w2[l]) + b2[l]
        x = _ln(x + ff, lnp[l][2:3], lnp[l][3:4])
    l = 2
    xs = jnp.concatenate([x[:, 0:1, :], x[:, NTOK + 1:NTOK + 2, :]], axis=1)
    kv = _mm(x, wqkv[l][:, 50:]) + bqkv[l][:, 50:]
    q2 = _mm(xs, wqkv[l][:, :50]) + bqkv[l][:, :50]
    ao = jnp.stack([_attn_one(q2[i], kv[i, :, :50], kv[i, :, 50:])
                    for i in range(b)])
    xs = _ln(xs + _mm(ao, wo[l]) + bo[l], lnp[l][0:1], lnp[l][1:2])
    ff = _mm(jnp.maximum(_mm(xs, w1[l]) + b1[l], 0.0), w2[l]) + b2[l]
    xs = _ln(xs + ff, lnp[l][2:3], lnp[l][3:4])
    wnt = wnt_ref[...]
    as_ref[...] = _arc(xs[:, 0, :], wnt, 0)
    af_ref[...] = _arc(xs[:, 1, :], wnt, 1)


def _run_transformer(tok, params):
    p = params
    base = jnp.concatenate(
        [p['cls_sig'].reshape(1, NZ), jnp.zeros((NTOK, NZ), F32),
         p['cls_freq'].reshape(1, NZ), jnp.zeros((NTOK, NZ), F32)],
        axis=0) + p['pos']
    tf = p['tf']
    wqkv = jnp.stack([jnp.concatenate(
        [lp['Wq'], lp['Wk'], lp['Wv']], axis=1) for lp in tf])
    bqkv = jnp.stack([jnp.concatenate(
        [lp['bq'], lp['bk'], lp['bv']])[None, :] for lp in tf])
    wo = jnp.stack([lp['Wo'] for lp in tf])
    bo = jnp.stack([lp['bo'][None, :] for lp in tf])
    w1 = jnp.stack([lp['W1'] for lp in tf])
    b1 = jnp.stack([lp['b1'][None, :] for lp in tf])
    w2 = jnp.stack([lp['W2'] for lp in tf])
    b2 = jnp.stack([lp['b2'][None, :] for lp in tf])
    lnp = jnp.stack([jnp.stack([lp['ln1_g'], lp['ln1_b'],
                                lp['ln2_g'], lp['ln2_b']]) for lp in tf])
    wn = p['arc_W']
    wnt = (wn / (jnp.linalg.norm(wn, axis=-1, keepdims=True) + 1e-12)).T

    nb = 256 // TB
    full = lambda a: pl.BlockSpec(a.shape, lambda i: (0,) * a.ndim)
    outs = pl.pallas_call(
        _tf_body,
        grid=(nb,),
        in_specs=[pl.BlockSpec((TB, 128, 50), lambda i: (i, 0, 0)),
                  pl.BlockSpec((TB, 128, 50), lambda i: (nb + i, 0, 0)),
                  full(base), full(wqkv), full(bqkv), full(wo), full(bo),
                  full(w1), full(b1), full(w2), full(b2), full(lnp),
                  full(wnt)],
        out_specs=[pl.BlockSpec((TB, 50), lambda i: (i, 0)),
                   pl.BlockSpec((TB, 50), lambda i: (i, 0))],
        out_shape=[jax.ShapeDtypeStruct((256, 50), F32),
                   jax.ShapeDtypeStruct((256, 50), F32)],
        compiler_params=pltpu.CompilerParams(
            dimension_semantics=("arbitrary",),
            vmem_limit_bytes=100 * 1024 * 1024),
        name="tf3arc",
    )(tok, tok, base, wqkv, bqkv, wo, bo, w1, b1, w2, b2, lnp, wnt)
    return outs


def _dec_body(as_ref, af_ref, w03, we1, wo1, we2, wo2, we3, wo3,
              we4, wo4, we5, wo5, wbd_ref, out_ref):
    wes = (we1, we2, we3, we4, we5)
    wos = (wo1, wo2, wo3, wo4, wo5)
    db = as_ref.shape[1]
    z = jnp.concatenate([as_ref[0], af_ref[0]], axis=0)
    a = lax.dot_general(z, w03[...], (((1,), (0,)), ((), ())),
                        preferred_element_type=F32)
    a = jnp.maximum(a, 0.0)
    c = 256
    for l in range(5):
        we, wo_ = wes[l][...], wos[l][...]
        s2 = 1 << l
        aext = jnp.concatenate(
            [_sd(a[:, :, -c:]), a, _su(a[:, :, :c])], axis=-1)
        pieces = []
        for p in range(s2):
            pieces.append(_mm(aext[:, :, p * c:(p + 2) * c], we))
            pieces.append(_mm(aext[:, :, (p + 1) * c:(p + 3) * c], wo_))
        a = jnp.maximum(jnp.concatenate(pieces, axis=-1), 0.0)
        c = we.shape[1]
    aext = jnp.concatenate(
        [_sd(a[:, :, -c:]), a, _su(a[:, :, :c])], axis=-1)
    cats = []
    for p in range(32):
        cats.append(aext[:, :, p * c:(p + 2) * c])
        cats.append(aext[:, :, (p + 1) * c:(p + 3) * c])
    cat = jnp.concatenate(cats, axis=-1)
    g = jnp.tanh(_mm(cat, wbd_ref[...]))
    out_ref[...] = 0.5 * (g[:db] + g[db:])


def _run_decoder(arc_s, arc_f, params):
    dws = params['dec']
    w03 = dws[0].transpose(0, 2, 1)
    wargs, wspecs = [], []
    full = lambda a: pl.BlockSpec(a.shape, lambda i: (0,) * a.ndim)
    for l in range(1, 6):
        w = dws[l]
        we = jnp.concatenate([w[:, :, 3], w[:, :, 1]], axis=0)
        wo_ = jnp.concatenate([w[:, :, 2], w[:, :, 0]], axis=0)
        wargs += [we, wo_]
        wspecs += [full(we), full(wo_)]
    w6 = dws[6]
    we6 = jnp.concatenate([w6[:, 0, 3], w6[:, 0, 1]])
    wo6 = jnp.concatenate([w6[:, 0, 2], w6[:, 0, 0]])
    base = jnp.stack([we6 if q % 2 == 0 else wo6 for q in range(64)])
    wbd = (base[:, :, None] * jnp.eye(64, dtype=F32)[:, None, :]
           ).reshape(2048, 64)
    nb = 256 // DB
    out = pl.pallas_call(
        _dec_body,
        grid=(nb,),
        in_specs=[pl.BlockSpec((1, DB, 50), lambda i: (i, 0, 0)),
                  pl.BlockSpec((1, DB, 50), lambda i: (i, 0, 0)),
                  full(w03)] + wspecs + [full(wbd)],
        out_specs=pl.BlockSpec((DB, 128, 64), lambda i: (i, 0, 0)),
        out_shape=jax.ShapeDtypeStruct((256, 128, 64), F32),
        compiler_params=pltpu.CompilerParams(
            dimension_semantics=("arbitrary",),
            vmem_limit_bytes=100 * 1024 * 1024),
        name="dec7",
    )(arc_s.reshape(nb, DB, 50), arc_f.reshape(nb, DB, 50), w03, *wargs, wbd)
    return out


def kernel(x_sig, x_freq, params):
    tok = _run_encoders(x_sig, x_freq, params)
    arc_s, arc_f = _run_transformer(tok, params)
    dec = _run_decoder(arc_s, arc_f, params)
    gen = dec.reshape(256, 1, 8192)
    latent_signal = tok[:256].transpose(0, 2, 1)
    latent_freq = tok[256:].transpose(0, 2, 1)
    return gen, latent_signal, latent_freq


# speedup vs baseline: 1.0482x; 1.0482x over previous
import math

import jax
import jax.numpy as jnp
from jax import lax
from jax.experimental import pallas as pl
from jax.experimental.pallas import tpu as pltpu

F32 = jnp.float32
NZ, NTOK, SEQ = 50, 128, 258
NHEAD, DHEAD, NHID = 5, 10, 512
ENC_CHS = [1, 16, 32, 64, 64, 64, 50]
ARC_S = 30.0
COS_M, SIN_M = math.cos(0.5), math.sin(0.5)
TH = math.cos(math.pi - 0.5)
MM = math.sin(math.pi - 0.5) * 0.5
INV_SQRT_D = 1.0 / math.sqrt(float(DHEAD))

EB = 16
TB = 4
DB = 8


def _mm(a, b):
    return lax.dot_general(a, b, (((a.ndim - 1,), (0,)), ((), ())),
                           preferred_element_type=F32)


def _sd(p):
    z = jnp.zeros((p.shape[0], 1, p.shape[2]), F32)
    return jnp.concatenate([z, p[:, :-1, :]], axis=1)


def _su(p):
    z = jnp.zeros((p.shape[0], 1, p.shape[2]), F32)
    return jnp.concatenate([p[:, 1:, :], z], axis=1)


def _enc_body(x_ref, w0, w1, w2, w3, w4, w5, out_ref):
    ws = (w0, w1, w2, w3, w4, w5)
    a = x_ref[...]
    c = 1
    for li in range(6):
        w = ws[li][0]
        s2 = a.shape[2] // (2 * c)
        aext = jnp.concatenate(
            [_sd(a[:, :, -c:]), a, _su(a[:, :, :c])], axis=-1)
        pieces = [_mm(aext[:, :, 2 * q * c:(2 * q + 4) * c], w)
                  for q in range(s2)]
        a = pieces[0] if s2 == 1 else jnp.concatenate(pieces, axis=-1)
        if li < 5:
            a = jnp.where(a > 0, a, 0.2 * a)
        c = w.shape[1]
    out_ref[...] = a


def _run_encoders(x_sig, x_freq, params):
    xin = jnp.concatenate([x_sig.reshape(-1, 8192),
                           x_freq.reshape(-1, 8192)],
                          axis=0).reshape(512, 128, 64)
    wstk = []
    for i in range(6):
        co = ENC_CHS[i + 1]
        wstk.append(jnp.stack(
            [params['enc_sig'][i].transpose(2, 1, 0).reshape(-1, co),
             params['enc_freq'][i].transpose(2, 1, 0).reshape(-1, co)]))
    nb = 256 // EB
    w_specs = [pl.BlockSpec((1,) + w.shape[1:], lambda s, i: (s, 0, 0))
               for w in wstk]
    return pl.pallas_call(
        _enc_body,
        grid=(2, nb),
        in_specs=[pl.BlockSpec((EB, 128, 64), lambda s, i: (s * nb + i, 0, 0))]
                 + w_specs,
        out_specs=pl.BlockSpec((EB, 128, 50), lambda s, i: (s * nb + i, 0, 0)),
        out_shape=jax.ShapeDtypeStruct((512, 128, 50), F32),
        compiler_params=pltpu.CompilerParams(
            dimension_semantics=("arbitrary", "arbitrary"),
            vmem_limit_bytes=100 * 1024 * 1024),
        name="enc6",
    )(xin, *wstk)


def _ln(x, g, bb):
    m = jnp.mean(x, -1, keepdims=True)
    d = x - m
    v = jnp.mean(d * d, -1, keepdims=True)
    return d * lax.rsqrt(v + 1e-5) * g + bb


def _attn_one(qrows, krows, vrows):
    outs = []
    for h in range(NHEAD):
        q = qrows[:, h * DHEAD:(h + 1) * DHEAD]
        k = krows[:, h * DHEAD:(h + 1) * DHEAD]
        v = vrows[:, h * DHEAD:(h + 1) * DHEAD]
        s = lax.dot_general(q, k, (((1,), (1,)), ((), ())),
                            preferred_element_type=F32) * INV_SQRT_D
        e = jnp.exp(s - jnp.max(s, -1, keepdims=True))
        o = lax.dot_general(e, v, (((1,), (0,)), ((), ())),
                            preferred_element_type=F32)
        outs.append(o / jnp.sum(e, -1, keepdims=True))
    return jnp.concatenate(outs, axis=-1)


def _arc(v, wnt, label):
    n = jnp.sqrt(jnp.sum(v * v, -1, keepdims=True))
    xn = v / (n + 1e-12)
    cos = _mm(xn, wnt)
    sin = jnp.sqrt(jnp.clip(1.0 - cos * cos, 0.0, 1.0))
    phi = cos * COS_M - sin * SIN_M
    phi = jnp.where(cos > TH, phi, cos - MM)
    col = lax.broadcasted_iota(jnp.int32, cos.shape, 1)
    return ARC_S * jnp.where(col == label, phi, cos)


def _tf_body(sig_ref, frq_ref, base_ref, wqkv, bqkv, wo, bo,
             w1, b1, w2, b2, lnp, wnt_ref, as_ref, af_ref):
    b = sig_ref.shape[0]
    z1 = jnp.zeros((b, 1, NZ), F32)
    x = (jnp.concatenate([z1, sig_ref[...], z1, frq_ref[...]], axis=1)
         + base_ref[...][None])
    for l in range(2):
        qkv = _mm(x, wqkv[l]) + bqkv[l]
        ao = jnp.stack([_attn_one(qkv[i, :, :50], qkv[i, :, 50:100],
                                  qkv[i, :, 100:]) for i in range(b)])
        x = _ln(x + _mm(ao, wo[l]) + bo[l], lnp[l][0:1], lnp[l][1:2])
        ff = _mm(jnp.maximum(_mm(x, w1[l]) + b1[l], 0.0), w2[l]) + b2[l]
        x = _ln(x + ff, lnp[l][2:3], lnp[l][3:4])
    l = 2
    xs = jnp.concatenate([x[:, 0:1, :], x[:, NTOK + 1:NTOK + 2, :]], axis=1)
    kv = _mm(x, wqkv[l][:, 50:]) + bqkv[l][:, 50:]
    q2 = _mm(xs, wqkv[l][:, :50]) + bqkv[l][:, :50]
    ao = jnp.stack([_attn_one(q2[i], kv[i, :, :50], kv[i, :, 50:])
                    for i in range(b)])
    xs = _ln(xs + _mm(ao, wo[l]) + bo[l], lnp[l][0:1], lnp[l][1:2])
    ff = _mm(jnp.maximum(_mm(xs, w1[l]) + b1[l], 0.0), w2[l]) + b2[l]
    xs = _ln(xs + ff, lnp[l][2:3], lnp[l][3:4])
    wnt = wnt_ref[...]
    as_ref[0] = _arc(xs[:, 0, :], wnt, 0)
    af_ref[0] = _arc(xs[:, 1, :], wnt, 1)


def _run_transformer(tok, params):
    p = params
    base = jnp.concatenate(
        [p['cls_sig'].reshape(1, NZ), jnp.zeros((NTOK, NZ), F32),
         p['cls_freq'].reshape(1, NZ), jnp.zeros((NTOK, NZ), F32)],
        axis=0) + p['pos']
    tf = p['tf']
    wqkv = jnp.stack([jnp.concatenate(
        [lp['Wq'], lp['Wk'], lp['Wv']], axis=1) for lp in tf])
    bqkv = jnp.stack([jnp.concatenate(
        [lp['bq'], lp['bk'], lp['bv']])[None, :] for lp in tf])
    wo = jnp.stack([lp['Wo'] for lp in tf])
    bo = jnp.stack([lp['bo'][None, :] for lp in tf])
    w1 = jnp.stack([lp['W1'] for lp in tf])
    b1 = jnp.stack([lp['b1'][None, :] for lp in tf])
    w2 = jnp.stack([lp['W2'] for lp in tf])
    b2 = jnp.stack([lp['b2'][None, :] for lp in tf])
    lnp = jnp.stack([jnp.stack([lp['ln1_g'], lp['ln1_b'],
                                lp['ln2_g'], lp['ln2_b']]) for lp in tf])
    wn = p['arc_W']
    wnt = (wn / (jnp.linalg.norm(wn, axis=-1, keepdims=True) + 1e-12)).T

    nb = 256 // TB
    full = lambda a: pl.BlockSpec(a.shape, lambda i: (0,) * a.ndim)
    outs = pl.pallas_call(
        _tf_body,
        grid=(nb,),
        in_specs=[pl.BlockSpec((TB, 128, 50), lambda i: (i, 0, 0)),
                  pl.BlockSpec((TB, 128, 50), lambda i: (nb + i, 0, 0)),
                  full(base), full(wqkv), full(bqkv), full(wo), full(bo),
                  full(w1), full(b1), full(w2), full(b2), full(lnp),
                  full(wnt)],
        out_specs=[pl.BlockSpec((1, TB, 50), lambda i: (i, 0, 0)),
                   pl.BlockSpec((1, TB, 50), lambda i: (i, 0, 0))],
        out_shape=[jax.ShapeDtypeStruct((nb, TB, 50), F32),
                   jax.ShapeDtypeStruct((nb, TB, 50), F32)],
        compiler_params=pltpu.CompilerParams(
            dimension_semantics=("arbitrary",),
            vmem_limit_bytes=100 * 1024 * 1024),
        name="tf3arc",
    )(tok, tok, base, wqkv, bqkv, wo, bo, w1, b1, w2, b2, lnp, wnt)
    return outs[0].reshape(256, 50), outs[1].reshape(256, 50)


def _dec_body(as_ref, af_ref, w03, we1, wo1, we2, wo2, we3, wo3,
              we4, wo4, we5, wo5, wbd_ref, out_ref):
    wes = (we1, we2, we3, we4, we5)
    wos = (wo1, wo2, wo3, wo4, wo5)
    db = as_ref.shape[1]
    z = jnp.concatenate([as_ref[0], af_ref[0]], axis=0)
    a = lax.dot_general(z, w03[...], (((1,), (0,)), ((), ())),
                        preferred_element_type=F32)
    a = jnp.maximum(a, 0.0)
    c = 256
    for l in range(5):
        we, wo_ = wes[l][...], wos[l][...]
        s2 = 1 << l
        aext = jnp.concatenate(
            [_sd(a[:, :, -c:]), a, _su(a[:, :, :c])], axis=-1)
        pieces = []
        for p in range(s2):
            pieces.append(_mm(aext[:, :, p * c:(p + 2) * c], we))
            pieces.append(_mm(aext[:, :, (p + 1) * c:(p + 3) * c], wo_))
        a = jnp.maximum(jnp.concatenate(pieces, axis=-1), 0.0)
        c = we.shape[1]
    aext = jnp.concatenate(
        [_sd(a[:, :, -c:]), a, _su(a[:, :, :c])], axis=-1)
    cats = []
    for p in range(32):
        cats.append(aext[:, :, p * c:(p + 2) * c])
        cats.append(aext[:, :, (p + 1) * c:(p + 3) * c])
    cat = jnp.concatenate(cats, axis=-1)
    g = jnp.tanh(_mm(cat, wbd_ref[...]))
    out_ref[...] = 0.5 * (g[:db] + g[db:])


def _run_decoder(arc_s, arc_f, params):
    dws = params['dec']
    w03 = dws[0].transpose(0, 2, 1)
    wargs, wspecs = [], []
    full = lambda a: pl.BlockSpec(a.shape, lambda i: (0,) * a.ndim)
    for l in range(1, 6):
        w = dws[l]
        we = jnp.concatenate([w[:, :, 3], w[:, :, 1]], axis=0)
        wo_ = jnp.concatenate([w[:, :, 2], w[:, :, 0]], axis=0)
        wargs += [we, wo_]
        wspecs += [full(we), full(wo_)]
    w6 = dws[6]
    we6 = jnp.concatenate([w6[:, 0, 3], w6[:, 0, 1]])
    wo6 = jnp.concatenate([w6[:, 0, 2], w6[:, 0, 0]])
    base = jnp.stack([we6 if q % 2 == 0 else wo6 for q in range(64)])
    wbd = (base[:, :, None] * jnp.eye(64, dtype=F32)[:, None, :]
           ).reshape(2048, 64)
    nb = 256 // DB
    out = pl.pallas_call(
        _dec_body,
        grid=(nb,),
        in_specs=[pl.BlockSpec((1, DB, 50), lambda i: (i, 0, 0)),
                  pl.BlockSpec((1, DB, 50), lambda i: (i, 0, 0)),
                  full(w03)] + wspecs + [full(wbd)],
        out_specs=pl.BlockSpec((DB, 128, 64), lambda i: (i, 0, 0)),
        out_shape=jax.ShapeDtypeStruct((256, 128, 64), F32),
        compiler_params=pltpu.CompilerParams(
            dimension_semantics=("arbitrary",),
            vmem_limit_bytes=100 * 1024 * 1024),
        name="dec7",
    )(arc_s.reshape(nb, DB, 50), arc_f.reshape(nb, DB, 50), w03, *wargs, wbd)
    return out


def kernel(x_sig, x_freq, params):
    tok = _run_encoders(x_sig, x_freq, params)
    arc_s, arc_f = _run_transformer(tok, params)
    dec = _run_decoder(arc_s, arc_f, params)
    gen = dec.reshape(256, 1, 8192)
    latent_signal = tok[:256].transpose(0, 2, 1)
    latent_freq = tok[256:].transpose(0, 2, 1)
    return gen, latent_signal, latent_freq


# speedup vs baseline: 1.2651x; 1.2070x over previous
import math

import jax
import jax.numpy as jnp
from jax import lax
from jax.experimental import pallas as pl
from jax.experimental.pallas import tpu as pltpu

F32 = jnp.float32
NZ, NTOK, SEQ = 50, 128, 258
NHEAD, DHEAD, NHID = 5, 10, 512
ENC_CHS = [1, 16, 32, 64, 64, 64, 50]
ARC_S = 30.0
COS_M, SIN_M = math.cos(0.5), math.sin(0.5)
TH = math.cos(math.pi - 0.5)
MM = math.sin(math.pi - 0.5) * 0.5
INV_SQRT_D = 1.0 / math.sqrt(float(DHEAD))

EB = 16
TB = 4
DB = 8


def _mm(a, b):
    return lax.dot_general(a, b, (((a.ndim - 1,), (0,)), ((), ())),
                           preferred_element_type=F32)


def _sd(p):
    z = jnp.zeros((p.shape[0], 1, p.shape[2]), F32)
    return jnp.concatenate([z, p[:, :-1, :]], axis=1)


def _su(p):
    z = jnp.zeros((p.shape[0], 1, p.shape[2]), F32)
    return jnp.concatenate([p[:, 1:, :], z], axis=1)


def _enc_body(x_ref, w0, w1, w2, w3, w4, w5, out_ref):
    ws = (w0, w1, w2, w3, w4, w5)
    a = x_ref[...]
    c = 1
    for li in range(6):
        w = ws[li][0]
        s2 = a.shape[2] // (2 * c)
        aext = jnp.concatenate(
            [_sd(a[:, :, -c:]), a, _su(a[:, :, :c])], axis=-1)
        pieces = [_mm(aext[:, :, 2 * q * c:(2 * q + 4) * c], w)
                  for q in range(s2)]
        a = pieces[0] if s2 == 1 else jnp.concatenate(pieces, axis=-1)
        if li < 5:
            a = jnp.where(a > 0, a, 0.2 * a)
        c = w.shape[1]
    out_ref[...] = a


def _run_encoders(x_sig, x_freq, params):
    xin = jnp.concatenate([x_sig.reshape(-1, 8192),
                           x_freq.reshape(-1, 8192)],
                          axis=0).reshape(512, 128, 64)
    wstk = []
    for i in range(6):
        co = ENC_CHS[i + 1]
        wstk.append(jnp.stack(
            [params['enc_sig'][i].transpose(2, 1, 0).reshape(-1, co),
             params['enc_freq'][i].transpose(2, 1, 0).reshape(-1, co)]))
    nb = 256 // EB
    w_specs = [pl.BlockSpec((1,) + w.shape[1:], lambda s, i: (s, 0, 0))
               for w in wstk]
    return pl.pallas_call(
        _enc_body,
        grid=(2, nb),
        in_specs=[pl.BlockSpec((EB, 128, 64), lambda s, i: (s * nb + i, 0, 0))]
                 + w_specs,
        out_specs=pl.BlockSpec((EB, 128, 50), lambda s, i: (s * nb + i, 0, 0)),
        out_shape=jax.ShapeDtypeStruct((512, 128, 50), F32),
        compiler_params=pltpu.CompilerParams(
            dimension_semantics=("arbitrary", "arbitrary"),
            vmem_limit_bytes=100 * 1024 * 1024),
        name="enc6",
    )(xin, *wstk)


def _ln(x, g, bb):
    m = jnp.mean(x, -1, keepdims=True)
    d = x - m
    v = jnp.mean(d * d, -1, keepdims=True)
    return d * lax.rsqrt(v + 1e-5) * g + bb


def _attn_one(qrows, krows, vrows):
    outs = []
    for h in range(NHEAD):
        q = qrows[:, h * DHEAD:(h + 1) * DHEAD]
        k = krows[:, h * DHEAD:(h + 1) * DHEAD]
        v = vrows[:, h * DHEAD:(h + 1) * DHEAD]
        s = lax.dot_general(q, k, (((1,), (1,)), ((), ())),
                            preferred_element_type=F32) * INV_SQRT_D
        e = jnp.exp(s - jnp.max(s, -1, keepdims=True))
        o = lax.dot_general(e, v, (((1,), (0,)), ((), ())),
                            preferred_element_type=F32)
        outs.append(o / jnp.sum(e, -1, keepdims=True))
    return jnp.concatenate(outs, axis=-1)


def _arc(v, wnt, label):
    n = jnp.sqrt(jnp.sum(v * v, -1, keepdims=True))
    xn = v / (n + 1e-12)
    cos = _mm(xn, wnt)
    sin = jnp.sqrt(jnp.clip(1.0 - cos * cos, 0.0, 1.0))
    phi = cos * COS_M - sin * SIN_M
    phi = jnp.where(cos > TH, phi, cos - MM)
    col = lax.broadcasted_iota(jnp.int32, cos.shape, 1)
    return ARC_S * jnp.where(col == label, phi, cos)


def _attn_bd(q, k, v, segmask):
    colh = lax.broadcasted_iota(jnp.int32, (1, NZ), 1) // DHEAD
    kparts, vparts = [], []
    zp = jnp.zeros((384 - SEQ, NZ), F32)
    for h in range(NHEAD):
        msk = (colh == h).astype(F32)
        kparts.append(jnp.concatenate([k * msk, zp], axis=0))
        vparts.append(jnp.concatenate([v * msk, zp], axis=0))
    kbd = jnp.concatenate(kparts, axis=0)
    vbd = jnp.concatenate(vparts, axis=0)
    s = lax.dot_general(q, kbd, (((1,), (1,)), ((), ())),
                        preferred_element_type=F32) + segmask
    eparts, nparts = [], []
    for h in range(NHEAD):
        seg = s[:, 384 * h:384 * (h + 1)]
        e = jnp.exp(seg - jnp.max(seg, -1, keepdims=True))
        eparts.append(e)
        nparts.append(jnp.broadcast_to(jnp.sum(e, -1, keepdims=True),
                                       (SEQ, DHEAD)))
    ebd = jnp.concatenate(eparts, axis=-1)
    norm = jnp.concatenate(nparts, axis=-1)
    o = lax.dot_general(ebd, vbd, (((1,), (0,)), ((), ())),
                        preferred_element_type=F32)
    return o / norm


def _tf_body(sig_ref, frq_ref, base_ref, segmask_ref, wqkv, bqkv, wo, bo,
             w1, b1, w2, b2, lnp, wnt_ref, as_ref, af_ref):
    b = sig_ref.shape[0]
    z1 = jnp.zeros((b, 1, NZ), F32)
    segmask = segmask_ref[...]
    x = (jnp.concatenate([z1, sig_ref[...], z1, frq_ref[...]], axis=1)
         + base_ref[...][None])
    for l in range(2):
        qkv = _mm(x, wqkv[l]) + bqkv[l]
        ao = jnp.stack([_attn_bd(qkv[i, :, :50] * INV_SQRT_D,
                                 qkv[i, :, 50:100], qkv[i, :, 100:],
                                 segmask) for i in range(b)])
        x = _ln(x + _mm(ao, wo[l]) + bo[l], lnp[l][0:1], lnp[l][1:2])
        ff = _mm(jnp.maximum(_mm(x, w1[l]) + b1[l], 0.0), w2[l]) + b2[l]
        x = _ln(x + ff, lnp[l][2:3], lnp[l][3:4])
    l = 2
    xs = jnp.concatenate([x[:, 0:1, :], x[:, NTOK + 1:NTOK + 2, :]], axis=1)
    kv = _mm(x, wqkv[l][:, 50:]) + bqkv[l][:, 50:]
    q2 = _mm(xs, wqkv[l][:, :50]) + bqkv[l][:, :50]
    ao = jnp.stack([_attn_one(q2[i], kv[i, :, :50], kv[i, :, 50:])
                    for i in range(b)])
    xs = _ln(xs + _mm(ao, wo[l]) + bo[l], lnp[l][0:1], lnp[l][1:2])
    ff = _mm(jnp.maximum(_mm(xs, w1[l]) + b1[l], 0.0), w2[l]) + b2[l]
    xs = _ln(xs + ff, lnp[l][2:3], lnp[l][3:4])
    wnt = wnt_ref[...]
    as_ref[0] = _arc(xs[:, 0, :], wnt, 0)
    af_ref[0] = _arc(xs[:, 1, :], wnt, 1)


def _run_transformer(tok, params):
    p = params
    base = jnp.concatenate(
        [p['cls_sig'].reshape(1, NZ), jnp.zeros((NTOK, NZ), F32),
         p['cls_freq'].reshape(1, NZ), jnp.zeros((NTOK, NZ), F32)],
        axis=0) + p['pos']
    tf = p['tf']
    wqkv = jnp.stack([jnp.concatenate(
        [lp['Wq'], lp['Wk'], lp['Wv']], axis=1) for lp in tf])
    bqkv = jnp.stack([jnp.concatenate(
        [lp['bq'], lp['bk'], lp['bv']])[None, :] for lp in tf])
    wo = jnp.stack([lp['Wo'] for lp in tf])
    bo = jnp.stack([lp['bo'][None, :] for lp in tf])
    w1 = jnp.stack([lp['W1'] for lp in tf])
    b1 = jnp.stack([lp['b1'][None, :] for lp in tf])
    w2 = jnp.stack([lp['W2'] for lp in tf])
    b2 = jnp.stack([lp['b2'][None, :] for lp in tf])
    lnp = jnp.stack([jnp.stack([lp['ln1_g'], lp['ln1_b'],
                                lp['ln2_g'], lp['ln2_b']]) for lp in tf])
    wn = p['arc_W']
    wnt = (wn / (jnp.linalg.norm(wn, axis=-1, keepdims=True) + 1e-12)).T
    segmask = jnp.where((jnp.arange(5 * 384) % 384) < SEQ,
                        0.0, -1e30).astype(F32)[None, :]

    nb = 256 // TB
    full = lambda a: pl.BlockSpec(a.shape, lambda i: (0,) * a.ndim)
    outs = pl.pallas_call(
        _tf_body,
        grid=(nb,),
        in_specs=[pl.BlockSpec((TB, 128, 50), lambda i: (i, 0, 0)),
                  pl.BlockSpec((TB, 128, 50), lambda i: (nb + i, 0, 0)),
                  full(base), full(segmask), full(wqkv), full(bqkv),
                  full(wo), full(bo), full(w1), full(b1), full(w2),
                  full(b2), full(lnp), full(wnt)],
        out_specs=[pl.BlockSpec((1, TB, 50), lambda i: (i, 0, 0)),
                   pl.BlockSpec((1, TB, 50), lambda i: (i, 0, 0))],
        out_shape=[jax.ShapeDtypeStruct((nb, TB, 50), F32),
                   jax.ShapeDtypeStruct((nb, TB, 50), F32)],
        compiler_params=pltpu.CompilerParams(
            dimension_semantics=("arbitrary",),
            vmem_limit_bytes=100 * 1024 * 1024),
        name="tf3arc",
    )(tok, tok, base, segmask, wqkv, bqkv, wo, bo, w1, b1, w2, b2, lnp, wnt)
    return outs[0].reshape(256, 50), outs[1].reshape(256, 50)


def _dec_body(as_ref, af_ref, w03, we1, wo1, we2, wo2, we3, wo3,
              we4, wo4, we5, wo5, wbd_ref, out_ref):
    wes = (we1, we2, we3, we4, we5)
    wos = (wo1, wo2, wo3, wo4, wo5)
    db = as_ref.shape[1]
    z = jnp.concatenate([as_ref[0], af_ref[0]], axis=0)
    a = lax.dot_general(z, w03[...], (((1,), (0,)), ((), ())),
                        preferred_element_type=F32)
    a = jnp.maximum(a, 0.0)
    c = 256
    for l in range(5):
        we, wo_ = wes[l][...], wos[l][...]
        s2 = 1 << l
        aext = jnp.concatenate(
            [_sd(a[:, :, -c:]), a, _su(a[:, :, :c])], axis=-1)
        pieces = []
        for p in range(s2):
            pieces.append(_mm(aext[:, :, p * c:(p + 2) * c], we))
            pieces.append(_mm(aext[:, :, (p + 1) * c:(p + 3) * c], wo_))
        a = jnp.maximum(jnp.concatenate(pieces, axis=-1), 0.0)
        c = we.shape[1]
    aext = jnp.concatenate(
        [_sd(a[:, :, -c:]), a, _su(a[:, :, :c])], axis=-1)
    cats = []
    for p in range(32):
        cats.append(aext[:, :, p * c:(p + 2) * c])
        cats.append(aext[:, :, (p + 1) * c:(p + 3) * c])
    cat = jnp.concatenate(cats, axis=-1)
    g = jnp.tanh(_mm(cat, wbd_ref[...]))
    out_ref[...] = 0.5 * (g[:db] + g[db:])


def _run_decoder(arc_s, arc_f, params):
    dws = params['dec']
    w03 = dws[0].transpose(0, 2, 1)
    wargs, wspecs = [], []
    full = lambda a: pl.BlockSpec(a.shape, lambda i: (0,) * a.ndim)
    for l in range(1, 6):
        w = dws[l]
        we = jnp.concatenate([w[:, :, 3], w[:, :, 1]], axis=0)
        wo_ = jnp.concatenate([w[:, :, 2], w[:, :, 0]], axis=0)
        wargs += [we, wo_]
        wspecs += [full(we), full(wo_)]
    w6 = dws[6]
    we6 = jnp.concatenate([w6[:, 0, 3], w6[:, 0, 1]])
    wo6 = jnp.concatenate([w6[:, 0, 2], w6[:, 0, 0]])
    base = jnp.stack([we6 if q % 2 == 0 else wo6 for q in range(64)])
    wbd = (base[:, :, None] * jnp.eye(64, dtype=F32)[:, None, :]
           ).reshape(2048, 64)
    nb = 256 // DB
    out = pl.pallas_call(
        _dec_body,
        grid=(nb,),
        in_specs=[pl.BlockSpec((1, DB, 50), lambda i: (i, 0, 0)),
                  pl.BlockSpec((1, DB, 50), lambda i: (i, 0, 0)),
                  full(w03)] + wspecs + [full(wbd)],
        out_specs=pl.BlockSpec((DB, 128, 64), lambda i: (i, 0, 0)),
        out_shape=jax.ShapeDtypeStruct((256, 128, 64), F32),
        compiler_params=pltpu.CompilerParams(
            dimension_semantics=("arbitrary",),
            vmem_limit_bytes=100 * 1024 * 1024),
        name="dec7",
    )(arc_s.reshape(nb, DB, 50), arc_f.reshape(nb, DB, 50), w03, *wargs, wbd)
    return out


def kernel(x_sig, x_freq, params):
    tok = _run_encoders(x_sig, x_freq, params)
    arc_s, arc_f = _run_transformer(tok, params)
    dec = _run_decoder(arc_s, arc_f, params)
    gen = dec.reshape(256, 1, 8192)
    latent_signal = tok[:256].transpose(0, 2, 1)
    latent_freq = tok[256:].transpose(0, 2, 1)
    return gen, latent_signal, latent_freq
